# baseline (speedup 1.0000x reference)
import jax
import jax.numpy as jnp
from jax import lax
from jax.experimental import pallas as pl
from jax.experimental.pallas import tpu as pltpu
from jax.experimental.pallas import tpu_sc as plsc

RC2 = 6.25
BOX = 40.0
HALF_BOX = 20.0
N_ATOMS = 100000
N_EDGES = 3200000

NC, NS, L = 2, 16, 16
NW = NC * NS
WIN = 1024
BIG_WORKERS = 21
WINS_HI = 98
WINS_LO = 97
NWIN = 100
GROUPS = WIN // L
BLK = 128
NBLK = WIN // BLK
BGROUPS = BLK // L
ACC_ROWS = 100352
ROWS_PER_S = ACC_ROWS // NS


def _body(px_hbm, py_hbm, pz_hbm, ii_hbm, jj_hbm, eps_hbm, sig_hbm, zeros_hbm,
          fpart_hbm, partials_hbm,
          idx0, idx1, idx2, idx3,
          eps0, eps1, eps2, eps3, sig0, sig1, sig2, sig3,
          pcx0, pcx1, pcy0, pcy1, pcz0, pcz1,
          fcx0, fcx1, fcy0, fcy1, fcz0, fcz1,
          resb, px, py, pz, ax, ay, az,
          sl0, sl1, sl2, sl3, sg0, sg1, ss0, ss1):
    c = lax.axis_index("c")
    s = lax.axis_index("s")
    wid = s * NC + c

    idxs = [idx0, idx1, idx2, idx3]
    epss = [eps0, eps1, eps2, eps3]
    sigs = [sig0, sig1, sig2, sig3]
    pcx = [pcx0, pcx1]
    pcy = [pcy0, pcy1]
    pcz = [pcz0, pcz1]
    fcx = [fcx0, fcx1]
    fcy = [fcy0, fcy1]
    fcz = [fcz0, fcz1]
    slin = [sl0, sl1, sl2, sl3]
    sgat = [sg0, sg1]
    ssca = [ss0, ss1]

    def s_descs(k, m, bi):
        bsl = pl.ds(bi * BLK, BLK)
        bslj = pl.ds(WIN + bi * BLK, BLK)
        return [
            pltpu.make_async_copy(fcx[k].at[bsl],
                                  ax.at[idxs[m].at[bsl]], ssca[k]),
            pltpu.make_async_copy(fcy[k].at[bsl],
                                  ay.at[idxs[m].at[bsl]], ssca[k]),
            pltpu.make_async_copy(fcz[k].at[bsl],
                                  az.at[idxs[m].at[bsl]], ssca[k]),
            pltpu.make_async_copy(fcx[k].at[bslj],
                                  ax.at[idxs[m].at[bslj]], ssca[k]),
            pltpu.make_async_copy(fcy[k].at[bslj],
                                  ay.at[idxs[m].at[bslj]], ssca[k]),
            pltpu.make_async_copy(fcz[k].at[bslj],
                                  az.at[idxs[m].at[bslj]], ssca[k]),
        ]

    def drain_scatters(k, m, nb):
        def dwait(_, c):
            for d in s_descs(k, m, 0):
                d.wait()
            return c
        lax.fori_loop(0, nb, dwait, 0)

    rng = pl.ds(s * ROWS_PER_S, ROWS_PER_S)
    pltpu.sync_copy(px_hbm.at[rng], px.at[rng])
    pltpu.sync_copy(py_hbm.at[rng], py.at[rng])
    pltpu.sync_copy(pz_hbm.at[rng], pz.at[rng])
    pltpu.sync_copy(zeros_hbm.at[rng], ax.at[rng])
    pltpu.sync_copy(zeros_hbm.at[rng], ay.at[rng])
    pltpu.sync_copy(zeros_hbm.at[rng], az.at[rng])
    plsc.subcore_barrier()

    iota = lax.iota(jnp.int32, L)
    zf = jnp.zeros((L,), jnp.float32)
    nwin_real = jnp.where(wid < BIG_WORKERS, WINS_HI, WINS_LO)
    base_e = (wid * WINS_LO + jnp.minimum(wid, BIG_WORKERS)) * WIN

    def eoff(w):
        return jnp.minimum(base_e + w * WIN, N_EDGES - WIN)

    def l_descs(m, w):
        e = eoff(w)
        return [
            pltpu.make_async_copy(ii_hbm.at[pl.ds(e, WIN)],
                                  idxs[m].at[pl.ds(0, WIN)], slin[m]),
            pltpu.make_async_copy(jj_hbm.at[pl.ds(e, WIN)],
                                  idxs[m].at[pl.ds(WIN, WIN)], slin[m]),
            pltpu.make_async_copy(eps_hbm.at[pl.ds(e, WIN)], epss[m], slin[m]),
            pltpu.make_async_copy(sig_hbm.at[pl.ds(e, WIN)], sigs[m], slin[m]),
        ]

    def g_descs(k, m):
        return [
            pltpu.make_async_copy(px_hbm.at[idxs[m]], pcx[k], sgat[k]),
            pltpu.make_async_copy(py_hbm.at[idxs[m]], pcy[k], sgat[k]),
            pltpu.make_async_copy(pz_hbm.at[idxs[m]], pcz[k], sgat[k]),
        ]

    def win_body(w, k, m, accs, nb_prev):
        kn = 1 - k
        mn = (m + 1) % 4
        m2 = (m + 2) % 4

        for d in l_descs(mn, w + 1):
            d.wait()
        for d in g_descs(kn, mn):
            d.start()

        @pl.when(w >= 2)
        def _():
            drain_scatters(k, m2, nb_prev)

        for d in l_descs(m2, w + 2):
            d.start()

        for d in g_descs(k, m):
            d.wait()

        wf = zf + jnp.where(w < nwin_real, 1.0, 0.0)

        def mimage(d):
            o = jnp.where(d >= 0.0, HALF_BOX, -HALF_BOX)
            n = ((d + o) * (1.0 / BOX)).astype(jnp.int32).astype(jnp.float32)
            return d - BOX * n

        def blockfn(bi, bstate):
            baccs, nb = bstate[:8], bstate[8]

            def group(gg, a):
                en, vi, xx, xy, xz, yy, yz, zz, bh = a
                g = bi * BGROUPS + gg
                sli = pl.ds(g * L, L)
                slj = pl.ds(WIN + g * L, L)
                dx = mimage(pcx[k][sli] - pcx[k][slj])
                dy = mimage(pcy[k][sli] - pcy[k][slj])
                dz = mimage(pcz[k][sli] - pcz[k][slj])
                r2 = jnp.maximum(dx * dx + dy * dy + dz * dz, 1e-24)
                inv = 1.0 / r2
                ev = epss[m][sli] * wf
                sv = sigs[m][sli] * wf
                s2 = sv * sv * inv
                s6 = s2 * s2 * s2
                s12 = s6 * s6
                t = s12 + s12 - s6
                e24 = 24.0 * ev
                gco = e24 * t * inv
                within = r2 < RC2
                en = en + jnp.where(within, 4.0 * ev * (s12 - s6), zf)
                vi = vi + jnp.where(within, e24 * t, zf)
                fx = gco * dx
                fy = gco * dy
                fz = gco * dz
                xx = xx + fx * dx
                xy = xy + fx * dy
                xz = xz + fx * dz
                yy = yy + fy * dy
                yz = yz + fy * dz
                zz = zz + fz * dz
                fxm = jnp.where(within, fx, zf)
                fym = jnp.where(within, fy, zf)
                fzm = jnp.where(within, fz, zf)
                bh = bh + fxm * fxm + fym * fym + fzm * fzm
                fcx[k][sli] = fxm
                fcy[k][sli] = fym
                fcz[k][sli] = fzm
                fcx[k][slj] = -fxm
                fcy[k][slj] = -fym
                fcz[k][slj] = -fzm
                return (en, vi, xx, xy, xz, yy, yz, zz, bh)

            out = lax.fori_loop(0, BGROUPS, group,
                                tuple(baccs) + (jnp.zeros((L,), jnp.float32),))
            baccs, bh = out[:8], out[8]

            h = bh[0]
            for q in range(1, L):
                h = h + bh[q]

            @pl.when(h > 0.0)
            def _():
                for d in s_descs(k, m, bi):
                    d.start(add=True)

            nb = nb + jnp.where(h > 0.0, 1, 0)
            return tuple(baccs) + (nb,)

        out = lax.fori_loop(0, NBLK, blockfn, accs + (jnp.int32(0),))
        return out[:8], out[8]

    for d in l_descs(0, 0):
        d.start()
    for d in l_descs(1, 1):
        d.start()
    for d in l_descs(0, 0):
        d.wait()
    for d in g_descs(0, 0):
        d.start()

    init = tuple(jnp.zeros((L,), jnp.float32) for _ in range(8))

    def quad(q, state):
        w = q * 4
        accs, nb0, nb1 = state[:8], state[8], state[9]
        accs, nb0 = win_body(w, 0, 0, accs, nb0)
        accs, nb1 = win_body(w + 1, 1, 1, accs, nb1)
        accs, nb0 = win_body(w + 2, 0, 2, accs, nb0)
        accs, nb1 = win_body(w + 3, 1, 3, accs, nb1)
        return tuple(accs) + (nb0, nb1)

    state = lax.fori_loop(0, NWIN // 4, quad,
                          init + (jnp.int32(0), jnp.int32(0)))
    accs, nb0, nb1 = state[:8], state[8], state[9]

    for d in g_descs(0, 0):
        d.wait()
    for d in l_descs(1, NWIN + 1):
        d.wait()
    drain_scatters(0, 2, nb0)
    drain_scatters(1, 3, nb1)

    plsc.subcore_barrier()
    off = (c * 3) * ACC_ROWS + s * ROWS_PER_S
    pltpu.sync_copy(ax.at[rng], fpart_hbm.at[pl.ds(off, ROWS_PER_S)])
    pltpu.sync_copy(ay.at[rng], fpart_hbm.at[pl.ds(off + ACC_ROWS, ROWS_PER_S)])
    pltpu.sync_copy(az.at[rng],
                    fpart_hbm.at[pl.ds(off + 2 * ACC_ROWS, ROWS_PER_S)])

    for k, v in enumerate(accs):
        resb[pl.ds(k * L, L)] = v
    pltpu.sync_copy(resb, partials_hbm.at[pl.ds(wid * 128, 128)])


@jax.jit
def kernel(pos, edge_index, epsilon, sigma):
    pad = jnp.zeros((ACC_ROWS - N_ATOMS,), jnp.float32)
    pxh = jnp.concatenate([pos[:, 0], pad])
    pyh = jnp.concatenate([pos[:, 1], pad])
    pzh = jnp.concatenate([pos[:, 2], pad])
    zeros = jnp.zeros((ACC_ROWS,), jnp.float32)

    mesh = plsc.VectorSubcoreMesh(
        core_axis_name="c", subcore_axis_name="s",
        num_cores=NC, num_subcores=NS)
    run = pl.kernel(
        _body,
        out_type=(
            jax.ShapeDtypeStruct((NC * 3 * ACC_ROWS,), jnp.float32),
            jax.ShapeDtypeStruct((NW * 128,), jnp.float32),
        ),
        mesh=mesh,
        scratch_types=(
            [pltpu.VMEM((2 * WIN,), jnp.int32)] * 4
            + [pltpu.VMEM((WIN,), jnp.float32)] * 8
            + [pltpu.VMEM((2 * WIN,), jnp.float32)] * 12
            + [pltpu.VMEM((128,), jnp.float32)]
            + [pltpu.VMEM_SHARED((ACC_ROWS,), jnp.float32)] * 6
            + [pltpu.SemaphoreType.DMA] * 8
        ),
    )
    fpart, partials = run(pxh, pyh, pzh, edge_index[0], edge_index[1],
                          epsilon, sigma, zeros)

    fpart = fpart.reshape(NC, 3, ACC_ROWS)
    fsum = fpart[0] + fpart[1]
    forces = fsum[:, :N_ATOMS].T
    cs = partials.reshape(NW, 8, L).sum(axis=(0, 2))
    energy = cs[0]
    virial = cs[1]
    virial_tensor = jnp.array(
        [[cs[2], cs[3], cs[4]],
         [cs[3], cs[5], cs[6]],
         [cs[4], cs[6], cs[7]]], dtype=jnp.float32)
    return (energy, forces, virial, virial_tensor)


# speedup vs baseline: 2.5738x; 2.5738x over previous
import jax
import jax.numpy as jnp
from jax import lax
from jax.experimental import pallas as pl
from jax.experimental.pallas import tpu as pltpu
from jax.experimental.pallas import tpu_sc as plsc

RC2 = 6.25
BOX = 40.0
HALF_BOX = 20.0
MAGIC = 12582912.0
N_ATOMS = 100000
N_EDGES = 3200000

NC, NS, L = 2, 16, 16
NW = NC * NS
WIN = 1024
BIG_WORKERS = 21
WINS_HI = 98
WINS_LO = 97
NWIN = 100
GROUPS = WIN // L
BLK = 128
NBLK = WIN // BLK
BGROUPS = BLK // L
ACC_ROWS = 100352
ROWS_PER_S = ACC_ROWS // NS


def _body(px_hbm, py_hbm, pz_hbm, ii_hbm, jj_hbm, eps_hbm, sig_hbm, zeros_hbm,
          fpart_hbm, partials_hbm,
          idx0, idx1, idx2, idx3,
          eps0, eps1, eps2, eps3, sig0, sig1, sig2, sig3,
          pcx0, pcx1, pcy0, pcy1, pcz0, pcz1,
          fcx0, fcx1, fcy0, fcy1, fcz0, fcz1,
          resb, px, py, pz, ax, ay, az,
          sl0, sl1, sl2, sl3, sg0, sg1, ss0, ss1):
    c = lax.axis_index("c")
    s = lax.axis_index("s")
    wid = s * NC + c

    idxs = [idx0, idx1, idx2, idx3]
    epss = [eps0, eps1, eps2, eps3]
    sigs = [sig0, sig1, sig2, sig3]
    pcx = [pcx0, pcx1]
    pcy = [pcy0, pcy1]
    pcz = [pcz0, pcz1]
    fcx = [fcx0, fcx1]
    fcy = [fcy0, fcy1]
    fcz = [fcz0, fcz1]
    slin = [sl0, sl1, sl2, sl3]
    sgat = [sg0, sg1]
    ssca = [ss0, ss1]

    def s_descs(k, m, bi):
        bsl = pl.ds(bi * BLK, BLK)
        bslj = pl.ds(WIN + bi * BLK, BLK)
        return [
            pltpu.make_async_copy(fcx[k].at[bsl],
                                  ax.at[idxs[m].at[bsl]], ssca[k]),
            pltpu.make_async_copy(fcy[k].at[bsl],
                                  ay.at[idxs[m].at[bsl]], ssca[k]),
            pltpu.make_async_copy(fcz[k].at[bsl],
                                  az.at[idxs[m].at[bsl]], ssca[k]),
            pltpu.make_async_copy(fcx[k].at[bslj],
                                  ax.at[idxs[m].at[bslj]], ssca[k]),
            pltpu.make_async_copy(fcy[k].at[bslj],
                                  ay.at[idxs[m].at[bslj]], ssca[k]),
            pltpu.make_async_copy(fcz[k].at[bslj],
                                  az.at[idxs[m].at[bslj]], ssca[k]),
        ]

    def drain_scatters(k, m, nb):
        def dwait(_, c):
            for d in s_descs(k, m, 0):
                d.wait()
            return c
        lax.fori_loop(0, nb, dwait, 0)

    rng = pl.ds(s * ROWS_PER_S, ROWS_PER_S)
    pltpu.sync_copy(px_hbm.at[rng], px.at[rng])
    pltpu.sync_copy(py_hbm.at[rng], py.at[rng])
    pltpu.sync_copy(pz_hbm.at[rng], pz.at[rng])
    pltpu.sync_copy(zeros_hbm.at[rng], ax.at[rng])
    pltpu.sync_copy(zeros_hbm.at[rng], ay.at[rng])
    pltpu.sync_copy(zeros_hbm.at[rng], az.at[rng])
    plsc.subcore_barrier()

    iota = lax.iota(jnp.int32, L)
    zf = jnp.zeros((L,), jnp.float32)
    nwin_real = jnp.where(wid < BIG_WORKERS, WINS_HI, WINS_LO)
    base_e = (wid * WINS_LO + jnp.minimum(wid, BIG_WORKERS)) * WIN

    def eoff(w):
        return jnp.minimum(base_e + w * WIN, N_EDGES - WIN)

    def l_descs(m, w):
        e = eoff(w)
        return [
            pltpu.make_async_copy(ii_hbm.at[pl.ds(e, WIN)],
                                  idxs[m].at[pl.ds(0, WIN)], slin[m]),
            pltpu.make_async_copy(jj_hbm.at[pl.ds(e, WIN)],
                                  idxs[m].at[pl.ds(WIN, WIN)], slin[m]),
            pltpu.make_async_copy(eps_hbm.at[pl.ds(e, WIN)], epss[m], slin[m]),
            pltpu.make_async_copy(sig_hbm.at[pl.ds(e, WIN)], sigs[m], slin[m]),
        ]

    def g_descs(k, m):
        return [
            pltpu.make_async_copy(px.at[idxs[m]], pcx[k], sgat[k]),
            pltpu.make_async_copy(py.at[idxs[m]], pcy[k], sgat[k]),
            pltpu.make_async_copy(pz.at[idxs[m]], pcz[k], sgat[k]),
        ]

    def win_body(w, k, m, accs, nb_prev):
        kn = 1 - k
        mn = (m + 1) % 4
        m2 = (m + 2) % 4

        for d in l_descs(mn, w + 1):
            d.wait()
        for d in g_descs(kn, mn):
            d.start()

        @pl.when(w >= 2)
        def _():
            drain_scatters(k, m2, nb_prev)

        for d in l_descs(m2, w + 2):
            d.start()

        for d in g_descs(k, m):
            d.wait()

        wf = zf + jnp.where(w < nwin_real, 1.0, 0.0)

        def mimage(d):
            n = (d * (1.0 / BOX) + MAGIC) - MAGIC
            return d - BOX * n

        def blockfn(bi, bstate):
            baccs, nb = bstate[:8], bstate[8]

            def group(gg, a):
                en, vi, xx, xy, xz, yy, yz, zz, bh = a
                g = bi * BGROUPS + gg
                sli = pl.ds(g * L, L)
                slj = pl.ds(WIN + g * L, L)
                dx = mimage(pcx[k][sli] - pcx[k][slj])
                dy = mimage(pcy[k][sli] - pcy[k][slj])
                dz = mimage(pcz[k][sli] - pcz[k][slj])
                r2 = jnp.maximum(dx * dx + dy * dy + dz * dz, 1e-24)
                inv = 1.0 / r2
                ev = epss[m][sli] * wf
                sv = sigs[m][sli] * wf
                s2 = sv * sv * inv
                s6 = s2 * s2 * s2
                s12 = s6 * s6
                t = s12 + s12 - s6
                e24 = 24.0 * ev
                gco = e24 * t * inv
                within = r2 < RC2
                en = en + jnp.where(within, 4.0 * ev * (s12 - s6), zf)
                vi = vi + jnp.where(within, e24 * t, zf)
                fx = gco * dx
                fy = gco * dy
                fz = gco * dz
                xx = xx + fx * dx
                xy = xy + fx * dy
                xz = xz + fx * dz
                yy = yy + fy * dy
                yz = yz + fy * dz
                zz = zz + fz * dz
                fxm = jnp.where(within, fx, zf)
                fym = jnp.where(within, fy, zf)
                fzm = jnp.where(within, fz, zf)
                bh = bh + jnp.where(within, 1.0, zf)
                fcx[k][sli] = fxm
                fcy[k][sli] = fym
                fcz[k][sli] = fzm
                fcx[k][slj] = -fxm
                fcy[k][slj] = -fym
                fcz[k][slj] = -fzm
                return (en, vi, xx, xy, xz, yy, yz, zz, bh)

            out = lax.fori_loop(0, BGROUPS, group,
                                tuple(baccs) + (jnp.zeros((L,), jnp.float32),))
            baccs, bh = out[:8], out[8]

            h = bh[0]
            for q in range(1, L):
                h = h + bh[q]

            @pl.when(h > 0.0)
            def _():
                for d in s_descs(k, m, bi):
                    d.start(add=True)

            nb = nb + jnp.where(h > 0.0, 1, 0)
            return tuple(baccs) + (nb,)

        out = lax.fori_loop(0, NBLK, blockfn, accs + (jnp.int32(0),))
        return out[:8], out[8]

    for d in l_descs(0, 0):
        d.start()
    for d in l_descs(1, 1):
        d.start()
    for d in l_descs(0, 0):
        d.wait()
    for d in g_descs(0, 0):
        d.start()

    init = tuple(jnp.zeros((L,), jnp.float32) for _ in range(8))

    def quad(q, state):
        w = q * 4
        accs, nb0, nb1 = state[:8], state[8], state[9]
        accs, nb0 = win_body(w, 0, 0, accs, nb0)
        accs, nb1 = win_body(w + 1, 1, 1, accs, nb1)
        accs, nb0 = win_body(w + 2, 0, 2, accs, nb0)
        accs, nb1 = win_body(w + 3, 1, 3, accs, nb1)
        return tuple(accs) + (nb0, nb1)

    state = lax.fori_loop(0, NWIN // 4, quad,
                          init + (jnp.int32(0), jnp.int32(0)))
    accs, nb0, nb1 = state[:8], state[8], state[9]

    for d in g_descs(0, 0):
        d.wait()
    for d in l_descs(1, NWIN + 1):
        d.wait()
    drain_scatters(0, 2, nb0)
    drain_scatters(1, 3, nb1)

    plsc.subcore_barrier()
    off = (c * 3) * ACC_ROWS + s * ROWS_PER_S
    pltpu.sync_copy(ax.at[rng], fpart_hbm.at[pl.ds(off, ROWS_PER_S)])
    pltpu.sync_copy(ay.at[rng], fpart_hbm.at[pl.ds(off + ACC_ROWS, ROWS_PER_S)])
    pltpu.sync_copy(az.at[rng],
                    fpart_hbm.at[pl.ds(off + 2 * ACC_ROWS, ROWS_PER_S)])

    for k, v in enumerate(accs):
        resb[pl.ds(k * L, L)] = v
    pltpu.sync_copy(resb, partials_hbm.at[pl.ds(wid * 128, 128)])


@jax.jit
def kernel(pos, edge_index, epsilon, sigma):
    pad = jnp.zeros((ACC_ROWS - N_ATOMS,), jnp.float32)
    pxh = jnp.concatenate([pos[:, 0], pad])
    pyh = jnp.concatenate([pos[:, 1], pad])
    pzh = jnp.concatenate([pos[:, 2], pad])
    zeros = jnp.zeros((ACC_ROWS,), jnp.float32)

    mesh = plsc.VectorSubcoreMesh(
        core_axis_name="c", subcore_axis_name="s",
        num_cores=NC, num_subcores=NS)
    run = pl.kernel(
        _body,
        out_type=(
            jax.ShapeDtypeStruct((NC * 3 * ACC_ROWS,), jnp.float32),
            jax.ShapeDtypeStruct((NW * 128,), jnp.float32),
        ),
        mesh=mesh,
        scratch_types=(
            [pltpu.VMEM((2 * WIN,), jnp.int32)] * 4
            + [pltpu.VMEM((WIN,), jnp.float32)] * 8
            + [pltpu.VMEM((2 * WIN,), jnp.float32)] * 12
            + [pltpu.VMEM((128,), jnp.float32)]
            + [pltpu.VMEM_SHARED((ACC_ROWS,), jnp.float32)] * 6
            + [pltpu.SemaphoreType.DMA] * 8
        ),
    )
    fpart, partials = run(pxh, pyh, pzh, edge_index[0], edge_index[1],
                          epsilon, sigma, zeros)

    fpart = fpart.reshape(NC, 3, ACC_ROWS)
    fsum = fpart[0] + fpart[1]
    forces = fsum[:, :N_ATOMS].T
    cs = partials.reshape(NW, 8, L).sum(axis=(0, 2))
    energy = cs[0]
    virial = cs[1]
    virial_tensor = jnp.array(
        [[cs[2], cs[3], cs[4]],
         [cs[3], cs[5], cs[6]],
         [cs[4], cs[6], cs[7]]], dtype=jnp.float32)
    return (energy, forces, virial, virial_tensor)
